# Initial kernel scaffold; baseline (speedup 1.0000x reference)
import jax
import jax.numpy as jnp
from jax.experimental import pallas as pl


def kernel(x, edge_index, W_lin, b_lin, W1, b1, W2, b2, W3, b3):
    raise NotImplementedError("write your pallas kernel here")


# speedup vs baseline: 10.9286x; 10.9286x over previous
import functools

import jax
import jax.numpy as jnp
from jax import lax
from jax.experimental import pallas as pl
from jax.experimental.pallas import tpu as pltpu
from jax.experimental.pallas import tpu_sc as plsc

_NC = 2
_NS = 16
_NW = _NC * _NS
_C = 128
_BN = 1024


def _scatter_fn(npad, dk, nch, gather):
  rpt = npad // _NS
  mesh = plsc.VectorSubcoreMesh(
      core_axis_name="c", subcore_axis_name="s",
      num_cores=_NC, num_subcores=_NS)

  @functools.partial(
      pl.kernel,
      out_type=jax.ShapeDtypeStruct((_NC * npad, dk), jnp.float32),
      mesh=mesh,
      scratch_types=[
          pltpu.VMEM((nch, _C), jnp.int32),
          pltpu.VMEM((nch, _C), jnp.int32),
          pltpu.VMEM((_C, dk), jnp.float32),
          pltpu.VMEM_SHARED((npad, dk), jnp.float32),
          pltpu.SemaphoreType.DMA,
      ],
  )
  def scatter(table, srcw, dstw, zrows, out, src_v, dst_v, buf, acc, sem):
    cid = lax.axis_index("c")
    sid = lax.axis_index("s")
    wid = sid * _NC + cid
    pltpu.sync_copy(srcw.at[wid], src_v)
    pltpu.sync_copy(dstw.at[wid], dst_v)
    if not gather:
      pltpu.sync_copy(table, buf)
    pltpu.sync_copy(zrows, acc.at[pl.ds(sid * rpt, rpt)])
    plsc.subcore_barrier()

    def step(i, carry):
      if gather:
        pltpu.async_copy(table.at[src_v.at[i]], buf, sem).wait()
      pltpu.sync_copy(buf, acc.at[dst_v.at[i]], add=True)
      return carry

    lax.fori_loop(0, nch, step, 0)
    plsc.subcore_barrier()
    pltpu.sync_copy(acc.at[pl.ds(sid * rpt, rpt)],
                    out.at[pl.ds(cid * npad + sid * rpt, rpt)])

  return scatter


def _row_specs(d, *lane_counts):
  return [pl.BlockSpec((_BN, lc if lc else d), lambda i: (i, 0))
          for lc in lane_counts]


def _w_spec(d):
  return pl.BlockSpec((d, d), lambda i: (0, 0))


def _b_spec(d):
  return pl.BlockSpec((1, d), lambda i: (0, 0))


def _tc_first(npad, n, d):
  def body(x_ref, d0_ref, d1_ref, wl_ref, bl_ref, w1_ref, y_ref, dinv_ref):
    pid = pl.program_id(0)
    deg = d0_ref[:, 0:1] + d1_ref[:, 0:1] + 1.0
    rows = lax.broadcasted_iota(jnp.int32, (_BN, 1), 0) + pid * _BN
    dinv = jnp.where(rows < n, lax.rsqrt(deg), 0.0)
    h = jnp.dot(x_ref[...], wl_ref[...],
                preferred_element_type=jnp.float32) + bl_ref[...]
    y_ref[...] = jnp.dot(h, w1_ref[...],
                         preferred_element_type=jnp.float32) * dinv
    dinv_ref[...] = jnp.broadcast_to(dinv, (_BN, d))

  return pl.pallas_call(
      body,
      grid=(npad // _BN,),
      in_specs=_row_specs(d, 0, 0, 0) + [_w_spec(d), _b_spec(d), _w_spec(d)],
      out_specs=_row_specs(d, 0, 0),
      out_shape=[jax.ShapeDtypeStruct((npad, d), jnp.float32),
                 jax.ShapeDtypeStruct((npad, d), jnp.float32)],
  )


def _tc_mid(npad, d):
  def body(s0_ref, s1_ref, y_ref, dinv_ref, b_ref, w_ref, o_ref):
    dinv = dinv_ref[...]
    h = (s0_ref[...] + s1_ref[...] + y_ref[...]) * dinv + b_ref[...]
    h = jnp.maximum(h, 0.0)
    o_ref[...] = jnp.dot(h, w_ref[...],
                         preferred_element_type=jnp.float32) * dinv

  return pl.pallas_call(
      body,
      grid=(npad // _BN,),
      in_specs=_row_specs(d, 0, 0, 0, 0) + [_b_spec(d), _w_spec(d)],
      out_specs=_row_specs(d, 0)[0],
      out_shape=jax.ShapeDtypeStruct((npad, d), jnp.float32),
  )


def _tc_last(npad, d):
  def body(s0_ref, s1_ref, y_ref, dinv_ref, b_ref, o_ref):
    o_ref[...] = ((s0_ref[...] + s1_ref[...] + y_ref[...]) * dinv_ref[...]
                  + b_ref[...])

  return pl.pallas_call(
      body,
      grid=(npad // _BN,),
      in_specs=_row_specs(d, 0, 0, 0, 0) + [_b_spec(d)],
      out_specs=_row_specs(d, 0)[0],
      out_shape=jax.ShapeDtypeStruct((npad, d), jnp.float32),
  )


def kernel(x, edge_index, W_lin, b_lin, W1, b1, W2, b2, W3, b3):
  n, d = x.shape
  e = edge_index.shape[1]

  npad = -((n + 1) // -_BN) * _BN
  ew = _NW * _C
  epad = -(e // -ew) * ew
  nch = epad // ew

  fill = jnp.full((epad - e,), n, jnp.int32)
  srcw = jnp.concatenate([edge_index[0], fill]).reshape(_NW, nch, _C)
  dstw = jnp.concatenate([edge_index[1], fill]).reshape(_NW, nch, _C)
  xp = jnp.pad(x, ((0, npad - n), (0, 0)))
  ones_cd = jnp.ones((_C, d), jnp.float32)
  zd = jnp.zeros((npad // _NS, d), jnp.float32)
  bl, bb1, bb2, bb3 = (v.reshape(1, d) for v in (b_lin, b1, b2, b3))

  scat_ones = _scatter_fn(npad, d, nch, gather=False)
  scatd = _scatter_fn(npad, d, nch, gather=True)

  degf = scat_ones(ones_cd, srcw, dstw, zd)
  y1, dinv = _tc_first(npad, n, d)(xp, degf[:npad], degf[npad:],
                                   W_lin, bl, W1)
  s1 = scatd(y1, srcw, dstw, zd)
  y2 = _tc_mid(npad, d)(s1[:npad], s1[npad:], y1, dinv, bb1, W2)
  s2 = scatd(y2, srcw, dstw, zd)
  y3 = _tc_mid(npad, d)(s2[:npad], s2[npad:], y2, dinv, bb2, W3)
  s3 = scatd(y3, srcw, dstw, zd)
  out = _tc_last(npad, d)(s3[:npad], s3[npad:], y3, dinv, bb3)
  return out[:n]
